# baseline (speedup 1.0000x reference)
import functools

import jax
import jax.numpy as jnp
import numpy as np
from jax import lax
from jax.experimental import pallas as pl
from jax.experimental.pallas import tpu as pltpu
from jax.experimental.pallas import tpu_sc as plsc

NC, NS, L = 2, 16, 16
NW = NC * NS
D = 128
N0 = 10000
E = 320000
NP = 10240
S = NP // NS
RPT = D // NW
ECH = 2000
ET = E // NS
RG = 16
CNT_W = NW - 1

_i32 = jnp.int32
_f32 = jnp.float32


def _skey_const(x):
    b = np.float32(x).view(np.int32)
    return int(b) if b >= 0 else int(np.int32(-2147483648) - b)


_LO0 = _skey_const(-2.5)
_HI0 = _skey_const(1.5)


def _skey(v):
    b = plsc.bitcast(v, _i32)
    return jnp.where(b >= 0, b, jnp.int32(-2147483648) - b)


def _agg_body(xp, src, dst, aggT, cnt, xrows, accr, cntb, sb0, db0, sb1, db1,
              sem0, sem1):
    cid = lax.axis_index("c")
    sid = lax.axis_index("s")
    w = sid * NC + cid
    c0 = w * RPT

    for q in range(RPT // 2):
        pltpu.sync_copy(xp.at[pl.ds((w * 2 + q) * NP, NP)],
                        xrows.at[pl.ds(q * NP, NP)])

    zf = jnp.zeros((L,), _f32)

    @pl.loop(0, RPT * NP // L)
    def _zero(j):
        accr[pl.ds(j * L, L)] = zf

    @pl.when(w == CNT_W)
    def _zero_cnt():
        @pl.loop(0, NP // L)
        def _z2(j):
            cntb[pl.ds(j * L, L)] = zf

    nchunks = E // ECH
    sbufs = (sb0, sb1)
    dbufs = (db0, db1)
    sems = (sem0, sem1)

    cp0 = pltpu.async_copy(src.at[pl.ds(0, ECH)], sb0, sem0)
    cp0b = pltpu.async_copy(dst.at[pl.ds(0, ECH)], db0, sem0)
    cp1 = pltpu.async_copy(src.at[pl.ds(ECH, ECH)], sb1, sem1)
    cp1b = pltpu.async_copy(dst.at[pl.ds(ECH, ECH)], db1, sem1)

    ones = jnp.ones((L,), _f32)

    @pl.loop(0, nchunks)
    def _chunk(i):
        b = lax.rem(i, 2)
        for bi in range(2):
            @pl.when(b == bi)
            def _proc():
                sbuf, dbuf, sem = sbufs[bi], dbufs[bi], sems[bi]
                pltpu.make_async_copy(src.at[pl.ds(0, ECH)], sbuf, sem).wait()
                pltpu.make_async_copy(dst.at[pl.ds(0, ECH)], dbuf, sem).wait()

                @functools.partial(plsc.parallel_loop, 0, ECH // L, unroll=8)
                def _grp(g):
                    s16 = sbuf[pl.ds(g * L, L)]
                    d16 = dbuf[pl.ds(g * L, L)]
                    m = (s16 >= 0) & (d16 >= 0)
                    sc = jnp.maximum(s16, 0)
                    dc = jnp.maximum(d16, 0)
                    for q in range(RPT // 2):
                        wv = plsc.load_gather(xrows, [sc + (q * NP)])
                        fa = plsc.bitcast(lax.shift_left(wv, jnp.int32(16)),
                                          _f32)
                        fb = plsc.bitcast(wv & jnp.int32(-65536), _f32)
                        plsc.addupdate_scatter(accr, [dc + (2 * q * NP)], fa,
                                               mask=m)
                        plsc.addupdate_scatter(accr, [dc + ((2 * q + 1) * NP)],
                                               fb, mask=m)

                @pl.when(w == CNT_W)
                def _cnt():
                    @functools.partial(plsc.parallel_loop, 0, ECH // L,
                                       unroll=4)
                    def _cgrp(g):
                        s16 = sbuf[pl.ds(g * L, L)]
                        d16 = dbuf[pl.ds(g * L, L)]
                        m = (s16 >= 0) & (d16 >= 0)
                        dc = jnp.maximum(d16, 0)
                        plsc.addupdate_scatter(cntb, [dc], ones, mask=m)

                @pl.when(i + 2 < nchunks)
                def _refill():
                    off = (i + 2) * ECH
                    pltpu.async_copy(src.at[pl.ds(off, ECH)], sbuf, sem)
                    pltpu.async_copy(dst.at[pl.ds(off, ECH)], dbuf, sem)

    for r in range(RPT):
        pltpu.sync_copy(accr.at[pl.ds(r * NP, NP)],
                        aggT.at[pl.ds((c0 + r) * NP, NP)])

    @pl.when(w == CNT_W)
    def _wcnt():
        pltpu.sync_copy(cntb, cnt)


def _make_agg():
    mesh = plsc.VectorSubcoreMesh(core_axis_name="c", subcore_axis_name="s",
                                  num_cores=NC, num_subcores=NS)
    return pl.kernel(
        _agg_body,
        out_type=(jax.ShapeDtypeStruct((D * NP,), _f32),
                  jax.ShapeDtypeStruct((NP,), _f32)),
        mesh=mesh,
        scratch_types=[
            pltpu.VMEM((RPT // 2 * NP,), _i32),
            pltpu.VMEM((RPT * NP,), _f32),
            pltpu.VMEM((NP,), _f32),
            pltpu.VMEM((ECH,), _i32),
            pltpu.VMEM((ECH,), _i32),
            pltpu.VMEM((ECH,), _i32),
            pltpu.VMEM((ECH,), _i32),
            pltpu.SemaphoreType.DMA,
            pltpu.SemaphoreType.DMA,
        ],
        compiler_params=pltpu.CompilerParams(needs_layout_passes=False),
        name="sage_agg_sc",
    )


def _dense_body(aggT, cnt, xT, valid, WlT, bl, WrT, ph, hT, s):
    rcp = 1.0 / jnp.maximum(cnt[...], 1.0)
    mean = aggT[...] * rcp
    h = jnp.dot(WlT[...], mean, preferred_element_type=_f32)
    h = h + jnp.dot(WrT[...], xT[...], preferred_element_type=_f32)
    h = jnp.maximum(h + bl[...], 0.0)
    hT[...] = h
    sc = jnp.tanh(jnp.dot(ph[...], h, preferred_element_type=_f32))
    s[...] = jnp.where(valid[...] > 0.0, sc, -2.0)


def _make_dense(blk=512):
    grid = (NP // blk,)
    full = pl.BlockSpec((D, D), lambda i: (0, 0))
    colv = pl.BlockSpec((1, blk), lambda i: (0, i))
    mat = pl.BlockSpec((D, blk), lambda i: (0, i))
    return pl.pallas_call(
        _dense_body,
        grid=grid,
        in_specs=[mat, colv, mat, colv, full,
                  pl.BlockSpec((D, 1), lambda i: (0, 0)), full,
                  pl.BlockSpec((1, D), lambda i: (0, 0))],
        out_specs=[mat, colv],
        out_shape=(jax.ShapeDtypeStruct((D, NP), _f32),
                   jax.ShapeDtypeStruct((1, NP), _f32)),
        name="sage_dense_tc",
    )


def _topk_body(K, remap, s_in, hT, src, dst, *rest):
    if remap:
        (xnT, xp2, valid, src2, dst2, svals, keys, mbuf, posbuf, idxbuf,
         vbuf, cntrow, cntv, idxfull, hbuf, obuf, pbuf, ebuf, rbuf, hist,
         hmerge, hsum, lsem, ssem, cnt_sh, idx_sh, hist_sh) = rest
    else:
        (xnT, xp2, valid, svals, keys, mbuf, posbuf, idxbuf,
         vbuf, cntrow, cntv, idxfull, hbuf, obuf, pbuf, ebuf, rbuf, hist,
         hmerge, hsum, lsem, ssem, cnt_sh, idx_sh, hist_sh) = rest
        src2 = dst2 = None

    cid = lax.axis_index("c")
    w = lax.axis_index("s")
    base = w * S

    pltpu.sync_copy(s_in.at[pl.ds(base, S)], svals)

    @pl.loop(0, S // L)
    def _keys(j):
        keys[pl.ds(j * L, L)] = _skey(svals[pl.ds(j * L, L)])

    def _count_ge(t):
        tv = jnp.full((L,), t, _i32)

        def _acc(j, a):
            return a + jnp.where(keys[pl.ds(j * L, L)] >= tv, 1, 0)

        acc = pl.loop(0, S // L, init_carry=jnp.zeros((L,), _i32))(_acc)
        return jnp.sum(acc)

    MIN32 = jnp.int32(-2147483648)
    ones_i = jnp.ones((L,), _i32)
    zi = jnp.zeros((L,), _i32)
    kk = jnp.int32(K)
    pfx = jnp.int32(0)
    for rnd, shift in enumerate((24, 16, 8, 0)):
        for c in range(256 // L):
            hist[pl.ds(c * L, L)] = zi
        sh8 = shift + 8

        @pl.loop(0, S // L)
        def _hloc(g, rnd=rnd, shift=shift, sh8=sh8, pfx=pfx):
            kc = keys[pl.ds(g * L, L)]
            uk = kc ^ MIN32
            byte = lax.shift_right_logical(uk, jnp.int32(shift)) & 0xFF
            if rnd == 0:
                plsc.addupdate_scatter(hist, [byte], ones_i)
            else:
                mm = lax.shift_right_logical(uk, jnp.int32(sh8)) == pfx
                plsc.addupdate_scatter(hist, [byte], ones_i, mask=mm)

        pltpu.sync_copy(hist, hist_sh.at[pl.ds(w * 256, 256)])
        plsc.subcore_barrier()
        pltpu.sync_copy(hist_sh, hmerge)
        for c in range(256 // L):
            acc = hmerge[pl.ds(c * L, L)]
            for t in range(1, NS):
                acc = acc + hmerge[pl.ds(t * 256 + c * L, L)]
            hsum[pl.ds(c * L, L)] = acc
        plsc.subcore_barrier()

        found = jnp.int32(0)
        bsel = jnp.int32(0)
        above = jnp.int32(0)
        for c in range(256 // L - 1, -1, -1):
            hv = hsum[pl.ds(c * L, L)]
            rv = lax.rev(hv, (0,))
            inc = plsc.cumsum(rv)
            sel = (inc + above) >= kk
            npos = plsc.all_reduce_population_count(sel)
            ffs = plsc.all_reduce_ffs(sel)
            npos = npos[0] if getattr(npos, "ndim", 0) else npos
            ffs = ffs[0] if getattr(ffs, "ndim", 0) else ffs
            hit = (found == 0) & (npos > 0)
            bsel = jnp.where(hit, jnp.int32(c * L + (L - 1)) - ffs, bsel)
            found = jnp.where(npos > 0, jnp.int32(1), found)
            above = above + inc[L - 1]
        cgt = zi
        for c in range(256 // L):
            binv = lax.iota(_i32, L) + c * L
            cgt = cgt + jnp.where(binv > bsel, hsum[pl.ds(c * L, L)], 0)
        kk = kk - jnp.sum(cgt)
        pfx = lax.shift_left(pfx, jnp.int32(8)) | bsel
    tstar = pfx ^ MIN32

    cg = _count_ge(tstar + 1)
    ce = _count_ge(tstar) - cg
    cntrow[...] = jnp.full((L,), cg, _i32)
    pltpu.sync_copy(cntrow, cnt_sh.at[pl.ds(w * L, L)])
    cntrow[...] = jnp.full((L,), ce, _i32)
    pltpu.sync_copy(cntrow, cnt_sh.at[pl.ds((NS + w) * L, L)])
    plsc.subcore_barrier()
    pltpu.sync_copy(cnt_sh, cntv)
    plsc.subcore_barrier()

    c_gt_tot = cntv[pl.ds(0, L)][0]
    for u in range(1, NS):
        c_gt_tot = c_gt_tot + cntv[pl.ds(u * L, L)][0]
    eq_before = jnp.int32(0)
    for u in range(NS):
        eq_before = jnp.where(u < w, eq_before + cntv[pl.ds((NS + u) * L, L)][0],
                              eq_before)
    r_need = K - c_gt_tot
    ce_w = cntv[pl.ds((NS + w.astype(_i32)) * L, L)][0]
    take_eq = jnp.clip(r_need - eq_before, 0, ce_w)
    take_w = cntv[pl.ds(w * L, L)][0] + take_eq

    tv = jnp.full((L,), tstar, _i32)
    neg1 = jnp.full((L,), -1, _i32)

    def _sel(g, carry):
        run_sel, run_eq = carry
        kc = keys[pl.ds(g * L, L)]
        gt = kc > tv
        eq = kc == tv
        eqi = jnp.where(eq, 1, 0)
        eqx = plsc.cumsum(eqi) - eqi + run_eq
        sel = gt | (eq & (eqx < take_eq))
        seli = jnp.where(sel, 1, 0)
        selx = plsc.cumsum(seli) - seli + run_sel
        mbuf[pl.ds(g * L, L)] = seli
        posbuf[pl.ds(g * L, L)] = jnp.full((L,), run_sel, _i32)
        idxbuf[pl.ds(g * L, L)] = jnp.where(sel, base + selx, neg1)
        lane = lax.iota(_i32, L) + g * L
        vbuf[pl.ds(g * L, L)] = jnp.where(lane < take_w, 1.0, 0.0)
        return run_sel + jnp.sum(seli), run_eq + jnp.sum(eqi)

    pl.loop(0, S // L, init_carry=(jnp.int32(0), jnp.int32(0)))(_sel)

    @pl.when(cid == 0)
    def _wvalid():
        pltpu.sync_copy(vbuf, valid.at[pl.ds(base, S)])

    pltpu.sync_copy(idxbuf, idx_sh.at[pl.ds(base, S)])

    NRG = D // RG // NC

    def _row0(p):
        return (jnp.int32(NC) * p + cid) * RG

    def _fire_loads(p, half):
        r0 = _row0(p)
        for r in range(RG):
            pltpu.async_copy(hT.at[pl.ds((r0 + r) * NP + base, S)],
                             hbuf.at[pl.ds((half * RG + r) * S, S)], lsem)

    def _drain(sem, n):
        for _ in range(n):
            pltpu.make_async_copy(hT.at[pl.ds(0, S)],
                                  hbuf.at[pl.ds(0, S)], sem).wait()

    _fire_loads(jnp.int32(0), 0)
    for p in range(NRG):
        half = p % 2
        if p + 1 < NRG:
            _fire_loads(jnp.int32(p + 1), 1 - half)
        _drain(lsem, RG)
        if p >= 2:
            _drain(ssem, RG + RG // 2)

        @pl.loop(0, S // L)
        def _cmp(g, half=half):
            m = mbuf[pl.ds(g * L, L)] > 0
            pos = posbuf[pl.ds(g * L, L)][0]
            v = svals[pl.ds(g * L, L)]
            hvs = []
            for r in range(RG):
                hv = hbuf[pl.ds((half * RG + r) * S + g * L, L)] * v
                plsc.store_compressed(
                    obuf.at[pl.ds((half * RG + r) * (S + L) + pos, L)], hv,
                    mask=m)
                hvs.append(hv)
            for j in range(RG // 2):
                ba = plsc.bitcast(hvs[2 * j], _i32)
                bb = plsc.bitcast(hvs[2 * j + 1], _i32)
                word = (lax.shift_right_logical(ba, jnp.int32(16))
                        | (bb & jnp.int32(-65536)))
                plsc.store_compressed(
                    pbuf.at[pl.ds((half * (RG // 2) + j) * (S + L) + pos, L)],
                    word, mask=m)

        r0 = _row0(p)
        pr0 = (jnp.int32(NC) * p + cid) * (RG // 2)
        for r in range(RG):
            pltpu.async_copy(obuf.at[pl.ds((half * RG + r) * (S + L), S)],
                             xnT.at[pl.ds((r0 + r) * NP + base, S)], ssem)
        for j in range(RG // 2):
            pltpu.async_copy(
                pbuf.at[pl.ds((half * (RG // 2) + j) * (S + L), S)],
                xp2.at[pl.ds((pr0 + j) * NP + base, S)], ssem)
    _drain(ssem, 2 * (RG + RG // 2))

    plsc.subcore_barrier()
    pltpu.sync_copy(idx_sh, idxfull)

    if remap:
        ein = (src, dst)
        eout = (src2, dst2)
        nech = ET // ECH
        for c in range(NC):
            @pl.when(cid == c)
            def _remap(c=c):
                pltpu.async_copy(ein[c].at[pl.ds(w * ET, ECH)],
                                 ebuf.at[pl.ds(0, ECH)], lsem)
                for i in range(nech):
                    half = i % 2
                    if i + 1 < nech:
                        pltpu.async_copy(
                            ein[c].at[pl.ds(w * ET + (i + 1) * ECH, ECH)],
                            ebuf.at[pl.ds((1 - half) * ECH, ECH)], lsem)
                    pltpu.make_async_copy(ein[c].at[pl.ds(0, ECH)],
                                          ebuf.at[pl.ds(0, ECH)], lsem).wait()
                    if i >= 2:
                        pltpu.make_async_copy(
                            ein[c].at[pl.ds(0, ECH)],
                            ebuf.at[pl.ds(0, ECH)], ssem).wait()

                    @pl.loop(0, ECH // L)
                    def _egrp(g, half=half):
                        ev = ebuf[pl.ds(half * ECH + g * L, L)]
                        got = plsc.load_gather(idxfull, [jnp.maximum(ev, 0)])
                        rbuf[pl.ds(half * ECH + g * L, L)] = jnp.where(
                            ev >= 0, got, neg1)

                    pltpu.async_copy(rbuf.at[pl.ds(half * ECH, ECH)],
                                     eout[c].at[pl.ds(w * ET + i * ECH, ECH)],
                                     ssem)
                pltpu.make_async_copy(ein[c].at[pl.ds(0, ECH)],
                                      ebuf.at[pl.ds(0, ECH)], ssem).wait()
                pltpu.make_async_copy(ein[c].at[pl.ds(0, ECH)],
                                      ebuf.at[pl.ds(0, ECH)], ssem).wait()


def _make_topk(K, remap):
    mesh = plsc.VectorSubcoreMesh(core_axis_name="c", subcore_axis_name="s",
                                  num_cores=NC, num_subcores=NS)
    outs = [jax.ShapeDtypeStruct((D * NP,), _f32),
            jax.ShapeDtypeStruct((D // 2 * NP,), _i32),
            jax.ShapeDtypeStruct((NP,), _f32)]
    if remap:
        outs += [jax.ShapeDtypeStruct((E,), _i32),
                 jax.ShapeDtypeStruct((E,), _i32)]
    return pl.kernel(
        functools.partial(_topk_body, K, remap),
        out_type=tuple(outs),
        mesh=mesh,
        scratch_types=[
            pltpu.VMEM((S,), _f32),
            pltpu.VMEM((S,), _i32),
            pltpu.VMEM((S,), _i32),
            pltpu.VMEM((S,), _i32),
            pltpu.VMEM((S,), _i32),
            pltpu.VMEM((S,), _f32),
            pltpu.VMEM((L,), _i32),
            pltpu.VMEM((2 * NS * L,), _i32),
            pltpu.VMEM((NP,), _i32),
            pltpu.VMEM((2 * RG * S,), _f32),
            pltpu.VMEM((2 * RG * (S + L),), _f32),
            pltpu.VMEM((RG * (S + L),), _i32),
            pltpu.VMEM((2 * ECH,), _i32),
            pltpu.VMEM((2 * ECH,), _i32),
            pltpu.VMEM((256,), _i32),
            pltpu.VMEM((NS * 256,), _i32),
            pltpu.VMEM((256,), _i32),
            pltpu.SemaphoreType.DMA,
            pltpu.SemaphoreType.DMA,
            pltpu.VMEM_SHARED((2 * NS * L,), _i32),
            pltpu.VMEM_SHARED((NP,), _i32),
            pltpu.VMEM_SHARED((NS * 256,), _i32),
        ],
        compiler_params=pltpu.CompilerParams(needs_layout_passes=False),
        name="topk_sc",
    )


def _final_body(K3, xT, valid, W4T, b4, W5T, b5, out):
    xm = xT[...] * valid[...]
    g = jnp.sum(xm, axis=1, keepdims=True) / K3
    h = jnp.dot(W4T[...], g, preferred_element_type=_f32) + b4[...]
    h = jnp.maximum(h, 0.0)
    z = jnp.dot(W5T[...], h, preferred_element_type=_f32) + b5[...]
    m = jnp.max(z, axis=0, keepdims=True)
    e = jnp.exp(z - m)
    lse = jnp.log(jnp.sum(e, axis=0, keepdims=True)) + m
    out[...] = z - lse


def _make_final(K3):
    return pl.pallas_call(
        functools.partial(_final_body, float(K3)),
        out_shape=jax.ShapeDtypeStruct((10, 1), _f32),
        name="pool_mlp_tc",
    )


def kernel(x, edge_index, batch, Wl1, bl1, Wr1, p1, Wl2, bl2, Wr2, p2,
           Wl3, bl3, Wr3, p3, W4, b4, W5, b5):
    del batch
    n = x.shape[0]
    x2d = jnp.pad(x.T, ((0, 0), (0, NP - n)))
    xT = x2d.reshape(-1)
    bits = jax.lax.bitcast_convert_type(x2d, _i32)
    xp = ((jnp.right_shift(bits[0::2], 16) & 0xFFFF)
          | (bits[1::2] & jnp.int32(-65536))).reshape(-1)
    src = edge_index[0]
    dst = edge_index[1]
    valid = jnp.pad(jnp.ones((n,), _f32), (0, NP - n))

    agg = _make_agg()
    dense = _make_dense()
    ks = [int(np.ceil(0.8 * n))]
    ks.append(int(np.ceil(0.8 * ks[0])))
    ks.append(int(np.ceil(0.8 * ks[1])))

    layers = [(Wl1, bl1, Wr1, p1), (Wl2, bl2, Wr2, p2), (Wl3, bl3, Wr3, p3)]
    for i, (Wl, bl, Wr, p) in enumerate(layers):
        aggT, cnt = agg(xp, src, dst)
        ph = (p / jnp.linalg.norm(p)).reshape(1, D)
        hT, s = dense(aggT.reshape(D, NP), cnt.reshape(1, NP),
                      xT.reshape(D, NP), valid.reshape(1, NP),
                      Wl.T, bl.reshape(D, 1), Wr.T, ph)
        remap = i < 2
        tk = _make_topk(ks[i], remap)
        if remap:
            xT, xp, valid, src, dst = tk(s.reshape(NP), hT.reshape(-1),
                                         src, dst)
        else:
            xT, xp, valid = tk(s.reshape(NP), hT.reshape(-1), src, dst)

    out = _make_final(ks[2])(xT.reshape(D, NP), valid.reshape(1, NP), W4.T,
                             b4.reshape(64, 1), W5.T, b5.reshape(10, 1))
    return out.reshape(1, 10)


# speedup vs baseline: 1.0535x; 1.0535x over previous
import functools

import jax
import jax.numpy as jnp
import numpy as np
from jax import lax
from jax.experimental import pallas as pl
from jax.experimental.pallas import tpu as pltpu
from jax.experimental.pallas import tpu_sc as plsc

NC, NS, L = 2, 16, 16
NW = NC * NS
D = 128
N0 = 10000
E = 320000
NP = 10240
S = NP // NS
RPT = D // NW
ECH = 2000
ET = E // NS
RG = 16
CNT_W = NW - 1

_i32 = jnp.int32
_f32 = jnp.float32


def _skey_const(x):
    b = np.float32(x).view(np.int32)
    return int(b) if b >= 0 else int(np.int32(-2147483648) - b)


_LO0 = _skey_const(-2.5)
_HI0 = _skey_const(1.5)


def _skey(v):
    b = plsc.bitcast(v, _i32)
    return jnp.where(b >= 0, b, jnp.int32(-2147483648) - b)


def _agg_body(xT, src, dst, aggT, cnt, xrows, accr, cntb, sb0, db0, sb1, db1,
              sem0, sem1):
    cid = lax.axis_index("c")
    sid = lax.axis_index("s")
    w = sid * NC + cid
    c0 = w * RPT

    for r in range(RPT):
        pltpu.sync_copy(xT.at[pl.ds((c0 + r) * NP, NP)],
                        xrows.at[pl.ds(r * NP, NP)])

    zf = jnp.zeros((L,), _f32)

    @pl.loop(0, RPT * NP // L)
    def _zero(j):
        accr[pl.ds(j * L, L)] = zf

    @pl.when(w == CNT_W)
    def _zero_cnt():
        @pl.loop(0, NP // L)
        def _z2(j):
            cntb[pl.ds(j * L, L)] = zf

    nchunks = E // ECH
    sbufs = (sb0, sb1)
    dbufs = (db0, db1)
    sems = (sem0, sem1)

    cp0 = pltpu.async_copy(src.at[pl.ds(0, ECH)], sb0, sem0)
    cp0b = pltpu.async_copy(dst.at[pl.ds(0, ECH)], db0, sem0)
    cp1 = pltpu.async_copy(src.at[pl.ds(ECH, ECH)], sb1, sem1)
    cp1b = pltpu.async_copy(dst.at[pl.ds(ECH, ECH)], db1, sem1)

    ones = jnp.ones((L,), _f32)

    @pl.loop(0, nchunks)
    def _chunk(i):
        b = lax.rem(i, 2)
        for bi in range(2):
            @pl.when(b == bi)
            def _proc():
                sbuf, dbuf, sem = sbufs[bi], dbufs[bi], sems[bi]
                pltpu.make_async_copy(src.at[pl.ds(0, ECH)], sbuf, sem).wait()
                pltpu.make_async_copy(dst.at[pl.ds(0, ECH)], dbuf, sem).wait()

                @functools.partial(plsc.parallel_loop, 0, ECH // L, unroll=8)
                def _grp(g):
                    s16 = sbuf[pl.ds(g * L, L)]
                    d16 = dbuf[pl.ds(g * L, L)]
                    m = (s16 >= 0) & (d16 >= 0)
                    sc = jnp.maximum(s16, 0)
                    dc = jnp.maximum(d16, 0)
                    for r in range(RPT):
                        g1 = plsc.load_gather(xrows, [sc + (r * NP)])
                        plsc.addupdate_scatter(accr, [dc + (r * NP)], g1,
                                               mask=m)

                @pl.when(w == CNT_W)
                def _cnt():
                    @functools.partial(plsc.parallel_loop, 0, ECH // L,
                                       unroll=4)
                    def _cgrp(g):
                        s16 = sbuf[pl.ds(g * L, L)]
                        d16 = dbuf[pl.ds(g * L, L)]
                        m = (s16 >= 0) & (d16 >= 0)
                        dc = jnp.maximum(d16, 0)
                        plsc.addupdate_scatter(cntb, [dc], ones, mask=m)

                @pl.when(i + 2 < nchunks)
                def _refill():
                    off = (i + 2) * ECH
                    pltpu.async_copy(src.at[pl.ds(off, ECH)], sbuf, sem)
                    pltpu.async_copy(dst.at[pl.ds(off, ECH)], dbuf, sem)

    for r in range(RPT):
        pltpu.sync_copy(accr.at[pl.ds(r * NP, NP)],
                        aggT.at[pl.ds((c0 + r) * NP, NP)])

    @pl.when(w == CNT_W)
    def _wcnt():
        pltpu.sync_copy(cntb, cnt)


def _make_agg():
    mesh = plsc.VectorSubcoreMesh(core_axis_name="c", subcore_axis_name="s",
                                  num_cores=NC, num_subcores=NS)
    return pl.kernel(
        _agg_body,
        out_type=(jax.ShapeDtypeStruct((D * NP,), _f32),
                  jax.ShapeDtypeStruct((NP,), _f32)),
        mesh=mesh,
        scratch_types=[
            pltpu.VMEM((RPT * NP,), _f32),
            pltpu.VMEM((RPT * NP,), _f32),
            pltpu.VMEM((NP,), _f32),
            pltpu.VMEM((ECH,), _i32),
            pltpu.VMEM((ECH,), _i32),
            pltpu.VMEM((ECH,), _i32),
            pltpu.VMEM((ECH,), _i32),
            pltpu.SemaphoreType.DMA,
            pltpu.SemaphoreType.DMA,
        ],
        compiler_params=pltpu.CompilerParams(needs_layout_passes=False),
        name="sage_agg_sc",
    )


def _dense_body(aggT, cnt, xT, valid, WlT, bl, WrT, ph, hT, s):
    rcp = 1.0 / jnp.maximum(cnt[...], 1.0)
    mean = aggT[...] * rcp
    h = jnp.dot(WlT[...], mean, preferred_element_type=_f32)
    h = h + jnp.dot(WrT[...], xT[...], preferred_element_type=_f32)
    h = jnp.maximum(h + bl[...], 0.0)
    hT[...] = h
    sc = jnp.tanh(jnp.dot(ph[...], h, preferred_element_type=_f32))
    s[...] = jnp.where(valid[...] > 0.0, sc, -2.0)


def _make_dense(blk=512):
    grid = (NP // blk,)
    full = pl.BlockSpec((D, D), lambda i: (0, 0))
    colv = pl.BlockSpec((1, blk), lambda i: (0, i))
    mat = pl.BlockSpec((D, blk), lambda i: (0, i))
    return pl.pallas_call(
        _dense_body,
        grid=grid,
        in_specs=[mat, colv, mat, colv, full,
                  pl.BlockSpec((D, 1), lambda i: (0, 0)), full,
                  pl.BlockSpec((1, D), lambda i: (0, 0))],
        out_specs=[mat, colv],
        out_shape=(jax.ShapeDtypeStruct((D, NP), _f32),
                   jax.ShapeDtypeStruct((1, NP), _f32)),
        name="sage_dense_tc",
    )


def _topk_body(K, remap, s_in, hT, src, dst, *rest):
    if remap:
        (xnT, valid, src2, dst2, svals, keys, mbuf, posbuf, idxbuf,
         vbuf, cntrow, cntv, idxfull, hbuf, obuf, ebuf, rbuf, hist,
         hmerge, hsum, lsem, ssem, cnt_sh, idx_sh, hist_sh) = rest
    else:
        (xnT, valid, svals, keys, mbuf, posbuf, idxbuf,
         vbuf, cntrow, cntv, idxfull, hbuf, obuf, ebuf, rbuf, hist,
         hmerge, hsum, lsem, ssem, cnt_sh, idx_sh, hist_sh) = rest
        src2 = dst2 = None

    cid = lax.axis_index("c")
    w = lax.axis_index("s")
    base = w * S

    pltpu.sync_copy(s_in.at[pl.ds(base, S)], svals)

    @pl.loop(0, S // L)
    def _keys(j):
        keys[pl.ds(j * L, L)] = _skey(svals[pl.ds(j * L, L)])

    def _count_ge(t):
        tv = jnp.full((L,), t, _i32)

        def _acc(j, a):
            return a + jnp.where(keys[pl.ds(j * L, L)] >= tv, 1, 0)

        acc = pl.loop(0, S // L, init_carry=jnp.zeros((L,), _i32))(_acc)
        return jnp.sum(acc)

    MIN32 = jnp.int32(-2147483648)
    ones_i = jnp.ones((L,), _i32)
    zi = jnp.zeros((L,), _i32)
    kk = jnp.int32(K)
    pfx = jnp.int32(0)
    for rnd, shift in enumerate((24, 16, 8, 0)):
        for c in range(256 // L):
            hist[pl.ds(c * L, L)] = zi
        sh8 = shift + 8

        @pl.loop(0, S // L)
        def _hloc(g, rnd=rnd, shift=shift, sh8=sh8, pfx=pfx):
            kc = keys[pl.ds(g * L, L)]
            uk = kc ^ MIN32
            byte = lax.shift_right_logical(uk, jnp.int32(shift)) & 0xFF
            if rnd == 0:
                plsc.addupdate_scatter(hist, [byte], ones_i)
            else:
                mm = lax.shift_right_logical(uk, jnp.int32(sh8)) == pfx
                plsc.addupdate_scatter(hist, [byte], ones_i, mask=mm)

        pltpu.sync_copy(hist, hist_sh.at[pl.ds(w * 256, 256)])
        plsc.subcore_barrier()
        pltpu.sync_copy(hist_sh, hmerge)
        for c in range(256 // L):
            acc = hmerge[pl.ds(c * L, L)]
            for t in range(1, NS):
                acc = acc + hmerge[pl.ds(t * 256 + c * L, L)]
            hsum[pl.ds(c * L, L)] = acc
        plsc.subcore_barrier()

        found = jnp.int32(0)
        bsel = jnp.int32(0)
        above = jnp.int32(0)
        for c in range(256 // L - 1, -1, -1):
            hv = hsum[pl.ds(c * L, L)]
            rv = lax.rev(hv, (0,))
            inc = plsc.cumsum(rv)
            sel = (inc + above) >= kk
            npos = plsc.all_reduce_population_count(sel)
            ffs = plsc.all_reduce_ffs(sel)
            npos = npos[0] if getattr(npos, "ndim", 0) else npos
            ffs = ffs[0] if getattr(ffs, "ndim", 0) else ffs
            hit = (found == 0) & (npos > 0)
            bsel = jnp.where(hit, jnp.int32(c * L + (L - 1)) - ffs, bsel)
            found = jnp.where(npos > 0, jnp.int32(1), found)
            above = above + inc[L - 1]
        cgt = zi
        for c in range(256 // L):
            binv = lax.iota(_i32, L) + c * L
            cgt = cgt + jnp.where(binv > bsel, hsum[pl.ds(c * L, L)], 0)
        kk = kk - jnp.sum(cgt)
        pfx = lax.shift_left(pfx, jnp.int32(8)) | bsel
    tstar = pfx ^ MIN32

    cg = _count_ge(tstar + 1)
    ce = _count_ge(tstar) - cg
    cntrow[...] = jnp.full((L,), cg, _i32)
    pltpu.sync_copy(cntrow, cnt_sh.at[pl.ds(w * L, L)])
    cntrow[...] = jnp.full((L,), ce, _i32)
    pltpu.sync_copy(cntrow, cnt_sh.at[pl.ds((NS + w) * L, L)])
    plsc.subcore_barrier()
    pltpu.sync_copy(cnt_sh, cntv)
    plsc.subcore_barrier()

    c_gt_tot = cntv[pl.ds(0, L)][0]
    for u in range(1, NS):
        c_gt_tot = c_gt_tot + cntv[pl.ds(u * L, L)][0]
    eq_before = jnp.int32(0)
    for u in range(NS):
        eq_before = jnp.where(u < w, eq_before + cntv[pl.ds((NS + u) * L, L)][0],
                              eq_before)
    r_need = K - c_gt_tot
    ce_w = cntv[pl.ds((NS + w.astype(_i32)) * L, L)][0]
    take_eq = jnp.clip(r_need - eq_before, 0, ce_w)
    take_w = cntv[pl.ds(w * L, L)][0] + take_eq

    tv = jnp.full((L,), tstar, _i32)
    neg1 = jnp.full((L,), -1, _i32)

    def _sel(g, carry):
        run_sel, run_eq = carry
        kc = keys[pl.ds(g * L, L)]
        gt = kc > tv
        eq = kc == tv
        eqi = jnp.where(eq, 1, 0)
        eqx = plsc.cumsum(eqi) - eqi + run_eq
        sel = gt | (eq & (eqx < take_eq))
        seli = jnp.where(sel, 1, 0)
        selx = plsc.cumsum(seli) - seli + run_sel
        mbuf[pl.ds(g * L, L)] = seli
        posbuf[pl.ds(g * L, L)] = jnp.full((L,), run_sel, _i32)
        idxbuf[pl.ds(g * L, L)] = jnp.where(sel, base + selx, neg1)
        lane = lax.iota(_i32, L) + g * L
        vbuf[pl.ds(g * L, L)] = jnp.where(lane < take_w, 1.0, 0.0)
        return run_sel + jnp.sum(seli), run_eq + jnp.sum(eqi)

    pl.loop(0, S // L, init_carry=(jnp.int32(0), jnp.int32(0)))(_sel)

    @pl.when(cid == 0)
    def _wvalid():
        pltpu.sync_copy(vbuf, valid.at[pl.ds(base, S)])

    pltpu.sync_copy(idxbuf, idx_sh.at[pl.ds(base, S)])

    NRG = D // RG // NC

    def _row0(p):
        return (jnp.int32(NC) * p + cid) * RG

    def _fire_loads(p, half):
        r0 = _row0(p)
        for r in range(RG):
            pltpu.async_copy(hT.at[pl.ds((r0 + r) * NP + base, S)],
                             hbuf.at[pl.ds((half * RG + r) * S, S)], lsem)

    def _drain(sem, n):
        for _ in range(n):
            pltpu.make_async_copy(hT.at[pl.ds(0, S)],
                                  hbuf.at[pl.ds(0, S)], sem).wait()

    _fire_loads(jnp.int32(0), 0)
    for p in range(NRG):
        half = p % 2
        if p + 1 < NRG:
            _fire_loads(jnp.int32(p + 1), 1 - half)
        _drain(lsem, RG)
        if p >= 2:
            _drain(ssem, RG)

        @pl.loop(0, S // L)
        def _cmp(g, half=half):
            m = mbuf[pl.ds(g * L, L)] > 0
            pos = posbuf[pl.ds(g * L, L)][0]
            v = svals[pl.ds(g * L, L)]
            for r in range(RG):
                hv = hbuf[pl.ds((half * RG + r) * S + g * L, L)] * v
                plsc.store_compressed(
                    obuf.at[pl.ds((half * RG + r) * (S + L) + pos, L)], hv,
                    mask=m)

        r0 = _row0(p)
        for r in range(RG):
            pltpu.async_copy(obuf.at[pl.ds((half * RG + r) * (S + L), S)],
                             xnT.at[pl.ds((r0 + r) * NP + base, S)], ssem)
    _drain(ssem, 2 * RG)

    plsc.subcore_barrier()
    pltpu.sync_copy(idx_sh, idxfull)

    if remap:
        ein = (src, dst)
        eout = (src2, dst2)
        nech = ET // ECH
        for c in range(NC):
            @pl.when(cid == c)
            def _remap(c=c):
                pltpu.async_copy(ein[c].at[pl.ds(w * ET, ECH)],
                                 ebuf.at[pl.ds(0, ECH)], lsem)
                for i in range(nech):
                    half = i % 2
                    if i + 1 < nech:
                        pltpu.async_copy(
                            ein[c].at[pl.ds(w * ET + (i + 1) * ECH, ECH)],
                            ebuf.at[pl.ds((1 - half) * ECH, ECH)], lsem)
                    pltpu.make_async_copy(ein[c].at[pl.ds(0, ECH)],
                                          ebuf.at[pl.ds(0, ECH)], lsem).wait()
                    if i >= 2:
                        pltpu.make_async_copy(
                            ein[c].at[pl.ds(0, ECH)],
                            ebuf.at[pl.ds(0, ECH)], ssem).wait()

                    @pl.loop(0, ECH // L)
                    def _egrp(g, half=half):
                        ev = ebuf[pl.ds(half * ECH + g * L, L)]
                        got = plsc.load_gather(idxfull, [jnp.maximum(ev, 0)])
                        rbuf[pl.ds(half * ECH + g * L, L)] = jnp.where(
                            ev >= 0, got, neg1)

                    pltpu.async_copy(rbuf.at[pl.ds(half * ECH, ECH)],
                                     eout[c].at[pl.ds(w * ET + i * ECH, ECH)],
                                     ssem)
                pltpu.make_async_copy(ein[c].at[pl.ds(0, ECH)],
                                      ebuf.at[pl.ds(0, ECH)], ssem).wait()
                pltpu.make_async_copy(ein[c].at[pl.ds(0, ECH)],
                                      ebuf.at[pl.ds(0, ECH)], ssem).wait()


def _make_topk(K, remap):
    mesh = plsc.VectorSubcoreMesh(core_axis_name="c", subcore_axis_name="s",
                                  num_cores=NC, num_subcores=NS)
    outs = [jax.ShapeDtypeStruct((D * NP,), _f32),
            jax.ShapeDtypeStruct((NP,), _f32)]
    if remap:
        outs += [jax.ShapeDtypeStruct((E,), _i32),
                 jax.ShapeDtypeStruct((E,), _i32)]
    return pl.kernel(
        functools.partial(_topk_body, K, remap),
        out_type=tuple(outs),
        mesh=mesh,
        scratch_types=[
            pltpu.VMEM((S,), _f32),
            pltpu.VMEM((S,), _i32),
            pltpu.VMEM((S,), _i32),
            pltpu.VMEM((S,), _i32),
            pltpu.VMEM((S,), _i32),
            pltpu.VMEM((S,), _f32),
            pltpu.VMEM((L,), _i32),
            pltpu.VMEM((2 * NS * L,), _i32),
            pltpu.VMEM((NP,), _i32),
            pltpu.VMEM((2 * RG * S,), _f32),
            pltpu.VMEM((2 * RG * (S + L),), _f32),
            pltpu.VMEM((2 * ECH,), _i32),
            pltpu.VMEM((2 * ECH,), _i32),
            pltpu.VMEM((256,), _i32),
            pltpu.VMEM((NS * 256,), _i32),
            pltpu.VMEM((256,), _i32),
            pltpu.SemaphoreType.DMA,
            pltpu.SemaphoreType.DMA,
            pltpu.VMEM_SHARED((2 * NS * L,), _i32),
            pltpu.VMEM_SHARED((NP,), _i32),
            pltpu.VMEM_SHARED((NS * 256,), _i32),
        ],
        compiler_params=pltpu.CompilerParams(needs_layout_passes=False),
        name="topk_sc",
    )


def _final_body(K3, xT, valid, W4T, b4, W5T, b5, out):
    xm = xT[...] * valid[...]
    g = jnp.sum(xm, axis=1, keepdims=True) / K3
    h = jnp.dot(W4T[...], g, preferred_element_type=_f32) + b4[...]
    h = jnp.maximum(h, 0.0)
    z = jnp.dot(W5T[...], h, preferred_element_type=_f32) + b5[...]
    m = jnp.max(z, axis=0, keepdims=True)
    e = jnp.exp(z - m)
    lse = jnp.log(jnp.sum(e, axis=0, keepdims=True)) + m
    out[...] = z - lse


def _make_final(K3):
    return pl.pallas_call(
        functools.partial(_final_body, float(K3)),
        out_shape=jax.ShapeDtypeStruct((10, 1), _f32),
        name="pool_mlp_tc",
    )


def kernel(x, edge_index, batch, Wl1, bl1, Wr1, p1, Wl2, bl2, Wr2, p2,
           Wl3, bl3, Wr3, p3, W4, b4, W5, b5):
    del batch
    n = x.shape[0]
    xT = jnp.pad(x.T, ((0, 0), (0, NP - n))).reshape(-1)
    src = edge_index[0]
    dst = edge_index[1]
    valid = jnp.pad(jnp.ones((n,), _f32), (0, NP - n))

    agg = _make_agg()
    dense = _make_dense()
    ks = [int(np.ceil(0.8 * n))]
    ks.append(int(np.ceil(0.8 * ks[0])))
    ks.append(int(np.ceil(0.8 * ks[1])))

    layers = [(Wl1, bl1, Wr1, p1), (Wl2, bl2, Wr2, p2), (Wl3, bl3, Wr3, p3)]
    for i, (Wl, bl, Wr, p) in enumerate(layers):
        aggT, cnt = agg(xT, src, dst)
        ph = (p / jnp.linalg.norm(p)).reshape(1, D)
        hT, s = dense(aggT.reshape(D, NP), cnt.reshape(1, NP),
                      xT.reshape(D, NP), valid.reshape(1, NP),
                      Wl.T, bl.reshape(D, 1), Wr.T, ph)
        remap = i < 2
        tk = _make_topk(ks[i], remap)
        if remap:
            xT, valid, src, dst = tk(s.reshape(NP), hT.reshape(-1), src, dst)
        else:
            xT, valid = tk(s.reshape(NP), hT.reshape(-1), src, dst)

    out = _make_final(ks[2])(xT.reshape(D, NP), valid.reshape(1, NP), W4.T,
                             b4.reshape(64, 1), W5.T, b5.reshape(10, 1))
    return out.reshape(1, 10)


# speedup vs baseline: 1.1678x; 1.1085x over previous
import functools

import jax
import jax.numpy as jnp
import numpy as np
from jax import lax
from jax.experimental import pallas as pl
from jax.experimental.pallas import tpu as pltpu
from jax.experimental.pallas import tpu_sc as plsc

NC, NS, L = 2, 16, 16
NW = NC * NS
D = 128
N0 = 10000
E = 320000
NP = 10240
S = NP // NS
RPT = D // NW
ECH = 4000
ET = E // NS
RG = 16
CNT_W = NW - 1

_i32 = jnp.int32
_f32 = jnp.float32


def _skey_const(x):
    b = np.float32(x).view(np.int32)
    return int(b) if b >= 0 else int(np.int32(-2147483648) - b)


_LO0 = _skey_const(-2.5)
_HI0 = _skey_const(1.5)


def _skey(v):
    b = plsc.bitcast(v, _i32)
    return jnp.where(b >= 0, b, jnp.int32(-2147483648) - b)


def _agg_body(xT, src, dst, aggT, cnt, xrows, accr, cntb, sb0, db0, sb1, db1,
              sem0, sem1):
    cid = lax.axis_index("c")
    sid = lax.axis_index("s")
    w = sid * NC + cid
    c0 = w * RPT

    for r in range(RPT):
        pltpu.sync_copy(xT.at[pl.ds((c0 + r) * NP, NP)],
                        xrows.at[pl.ds(r * NP, NP)])

    zf = jnp.zeros((L,), _f32)

    @pl.loop(0, RPT * NP // L)
    def _zero(j):
        accr[pl.ds(j * L, L)] = zf

    @pl.when(w == CNT_W)
    def _zero_cnt():
        @pl.loop(0, NP // L)
        def _z2(j):
            cntb[pl.ds(j * L, L)] = zf

    nchunks = E // ECH
    sbufs = (sb0, sb1)
    dbufs = (db0, db1)
    sems = (sem0, sem1)

    cp0 = pltpu.async_copy(src.at[pl.ds(0, ECH)], sb0, sem0)
    cp0b = pltpu.async_copy(dst.at[pl.ds(0, ECH)], db0, sem0)
    cp1 = pltpu.async_copy(src.at[pl.ds(ECH, ECH)], sb1, sem1)
    cp1b = pltpu.async_copy(dst.at[pl.ds(ECH, ECH)], db1, sem1)

    ones = jnp.ones((L,), _f32)

    @pl.loop(0, nchunks)
    def _chunk(i):
        b = lax.rem(i, 2)
        for bi in range(2):
            @pl.when(b == bi)
            def _proc():
                sbuf, dbuf, sem = sbufs[bi], dbufs[bi], sems[bi]
                pltpu.make_async_copy(src.at[pl.ds(0, ECH)], sbuf, sem).wait()
                pltpu.make_async_copy(dst.at[pl.ds(0, ECH)], dbuf, sem).wait()

                @functools.partial(plsc.parallel_loop, 0, ECH // L, unroll=8)
                def _grp(g):
                    s16 = sbuf[pl.ds(g * L, L)]
                    d16 = dbuf[pl.ds(g * L, L)]
                    m = (s16 >= 0) & (d16 >= 0)
                    sc = jnp.maximum(s16, 0)
                    dc = jnp.maximum(d16, 0)
                    for r in range(RPT):
                        g1 = plsc.load_gather(xrows, [sc + (r * NP)])
                        plsc.addupdate_scatter(accr, [dc + (r * NP)], g1,
                                               mask=m)

                @pl.when(w == CNT_W)
                def _cnt():
                    @functools.partial(plsc.parallel_loop, 0, ECH // L,
                                       unroll=4)
                    def _cgrp(g):
                        s16 = sbuf[pl.ds(g * L, L)]
                        d16 = dbuf[pl.ds(g * L, L)]
                        m = (s16 >= 0) & (d16 >= 0)
                        dc = jnp.maximum(d16, 0)
                        plsc.addupdate_scatter(cntb, [dc], ones, mask=m)

                @pl.when(i + 2 < nchunks)
                def _refill():
                    off = (i + 2) * ECH
                    pltpu.async_copy(src.at[pl.ds(off, ECH)], sbuf, sem)
                    pltpu.async_copy(dst.at[pl.ds(off, ECH)], dbuf, sem)

    for r in range(RPT):
        pltpu.sync_copy(accr.at[pl.ds(r * NP, NP)],
                        aggT.at[pl.ds((c0 + r) * NP, NP)])

    @pl.when(w == CNT_W)
    def _wcnt():
        pltpu.sync_copy(cntb, cnt)


def _make_agg():
    mesh = plsc.VectorSubcoreMesh(core_axis_name="c", subcore_axis_name="s",
                                  num_cores=NC, num_subcores=NS)
    return pl.kernel(
        _agg_body,
        out_type=(jax.ShapeDtypeStruct((D * NP,), _f32),
                  jax.ShapeDtypeStruct((NP,), _f32)),
        mesh=mesh,
        scratch_types=[
            pltpu.VMEM((RPT * NP,), _f32),
            pltpu.VMEM((RPT * NP,), _f32),
            pltpu.VMEM((NP,), _f32),
            pltpu.VMEM((ECH,), _i32),
            pltpu.VMEM((ECH,), _i32),
            pltpu.VMEM((ECH,), _i32),
            pltpu.VMEM((ECH,), _i32),
            pltpu.SemaphoreType.DMA,
            pltpu.SemaphoreType.DMA,
        ],
        compiler_params=pltpu.CompilerParams(needs_layout_passes=False),
        name="sage_agg_sc",
    )


def _dense_body(aggT, cnt, xT, valid, WlT, bl, WrT, ph, hT, s):
    rcp = 1.0 / jnp.maximum(cnt[...], 1.0)
    mean = aggT[...] * rcp
    h = jnp.dot(WlT[...], mean, preferred_element_type=_f32)
    h = h + jnp.dot(WrT[...], xT[...], preferred_element_type=_f32)
    h = jnp.maximum(h + bl[...], 0.0)
    hT[...] = h
    sc = jnp.tanh(jnp.dot(ph[...], h, preferred_element_type=_f32))
    s[...] = jnp.where(valid[...] > 0.0, sc, -2.0)


def _make_dense(blk=512):
    grid = (NP // blk,)
    full = pl.BlockSpec((D, D), lambda i: (0, 0))
    colv = pl.BlockSpec((1, blk), lambda i: (0, i))
    mat = pl.BlockSpec((D, blk), lambda i: (0, i))
    return pl.pallas_call(
        _dense_body,
        grid=grid,
        in_specs=[mat, colv, mat, colv, full,
                  pl.BlockSpec((D, 1), lambda i: (0, 0)), full,
                  pl.BlockSpec((1, D), lambda i: (0, 0))],
        out_specs=[mat, colv],
        out_shape=(jax.ShapeDtypeStruct((D, NP), _f32),
                   jax.ShapeDtypeStruct((1, NP), _f32)),
        name="sage_dense_tc",
    )


def _topk_body(K, remap, s_in, hT, src, dst, *rest):
    if remap:
        (xnT, valid, src2, dst2, svals, keys, mbuf, posbuf, idxbuf,
         vbuf, cntrow, cntv, idxfull, hbuf, obuf, ebuf, rbuf, hist,
         hmerge, hsum, lsem, ssem, cnt_sh, idx_sh, hist_sh) = rest
    else:
        (xnT, valid, svals, keys, mbuf, posbuf, idxbuf,
         vbuf, cntrow, cntv, idxfull, hbuf, obuf, ebuf, rbuf, hist,
         hmerge, hsum, lsem, ssem, cnt_sh, idx_sh, hist_sh) = rest
        src2 = dst2 = None

    cid = lax.axis_index("c")
    w = lax.axis_index("s")
    base = w * S

    pltpu.sync_copy(s_in.at[pl.ds(base, S)], svals)

    @pl.loop(0, S // L)
    def _keys(j):
        keys[pl.ds(j * L, L)] = _skey(svals[pl.ds(j * L, L)])

    def _count_ge(t):
        tv = jnp.full((L,), t, _i32)

        def _acc(j, a):
            return a + jnp.where(keys[pl.ds(j * L, L)] >= tv, 1, 0)

        acc = pl.loop(0, S // L, init_carry=jnp.zeros((L,), _i32))(_acc)
        return jnp.sum(acc)

    MIN32 = jnp.int32(-2147483648)
    ones_i = jnp.ones((L,), _i32)
    zi = jnp.zeros((L,), _i32)
    kk = jnp.int32(K)
    pfx = jnp.int32(0)
    for rnd, shift in enumerate((24, 16, 8, 0)):
        for c in range(256 // L):
            hist[pl.ds(c * L, L)] = zi
        sh8 = shift + 8

        @pl.loop(0, S // L)
        def _hloc(g, rnd=rnd, shift=shift, sh8=sh8, pfx=pfx):
            kc = keys[pl.ds(g * L, L)]
            uk = kc ^ MIN32
            byte = lax.shift_right_logical(uk, jnp.int32(shift)) & 0xFF
            if rnd == 0:
                plsc.addupdate_scatter(hist, [byte], ones_i)
            else:
                mm = lax.shift_right_logical(uk, jnp.int32(sh8)) == pfx
                plsc.addupdate_scatter(hist, [byte], ones_i, mask=mm)

        pltpu.sync_copy(hist, hist_sh.at[pl.ds(w * 256, 256)])
        plsc.subcore_barrier()
        pltpu.sync_copy(hist_sh, hmerge)
        for c in range(256 // L):
            acc = hmerge[pl.ds(c * L, L)]
            for t in range(1, NS):
                acc = acc + hmerge[pl.ds(t * 256 + c * L, L)]
            hsum[pl.ds(c * L, L)] = acc
        plsc.subcore_barrier()

        found = jnp.int32(0)
        bsel = jnp.int32(0)
        above = jnp.int32(0)
        for c in range(256 // L - 1, -1, -1):
            hv = hsum[pl.ds(c * L, L)]
            rv = lax.rev(hv, (0,))
            inc = plsc.cumsum(rv)
            sel = (inc + above) >= kk
            npos = plsc.all_reduce_population_count(sel)
            ffs = plsc.all_reduce_ffs(sel)
            npos = npos[0] if getattr(npos, "ndim", 0) else npos
            ffs = ffs[0] if getattr(ffs, "ndim", 0) else ffs
            hit = (found == 0) & (npos > 0)
            bsel = jnp.where(hit, jnp.int32(c * L + (L - 1)) - ffs, bsel)
            found = jnp.where(npos > 0, jnp.int32(1), found)
            above = above + inc[L - 1]
        cgt = zi
        for c in range(256 // L):
            binv = lax.iota(_i32, L) + c * L
            cgt = cgt + jnp.where(binv > bsel, hsum[pl.ds(c * L, L)], 0)
        kk = kk - jnp.sum(cgt)
        pfx = lax.shift_left(pfx, jnp.int32(8)) | bsel
    tstar = pfx ^ MIN32

    cg = _count_ge(tstar + 1)
    ce = _count_ge(tstar) - cg
    cntrow[...] = jnp.full((L,), cg, _i32)
    pltpu.sync_copy(cntrow, cnt_sh.at[pl.ds(w * L, L)])
    cntrow[...] = jnp.full((L,), ce, _i32)
    pltpu.sync_copy(cntrow, cnt_sh.at[pl.ds((NS + w) * L, L)])
    plsc.subcore_barrier()
    pltpu.sync_copy(cnt_sh, cntv)
    plsc.subcore_barrier()

    c_gt_tot = cntv[pl.ds(0, L)][0]
    for u in range(1, NS):
        c_gt_tot = c_gt_tot + cntv[pl.ds(u * L, L)][0]
    eq_before = jnp.int32(0)
    for u in range(NS):
        eq_before = jnp.where(u < w, eq_before + cntv[pl.ds((NS + u) * L, L)][0],
                              eq_before)
    r_need = K - c_gt_tot
    ce_w = cntv[pl.ds((NS + w.astype(_i32)) * L, L)][0]
    take_eq = jnp.clip(r_need - eq_before, 0, ce_w)
    take_w = cntv[pl.ds(w * L, L)][0] + take_eq

    tv = jnp.full((L,), tstar, _i32)
    neg1 = jnp.full((L,), -1, _i32)

    def _sel(g, carry):
        run_sel, run_eq = carry
        kc = keys[pl.ds(g * L, L)]
        gt = kc > tv
        eq = kc == tv
        eqi = jnp.where(eq, 1, 0)
        eqx = plsc.cumsum(eqi) - eqi + run_eq
        sel = gt | (eq & (eqx < take_eq))
        seli = jnp.where(sel, 1, 0)
        selx = plsc.cumsum(seli) - seli + run_sel
        mbuf[pl.ds(g * L, L)] = seli
        posbuf[pl.ds(g * L, L)] = jnp.full((L,), run_sel, _i32)
        idxbuf[pl.ds(g * L, L)] = jnp.where(sel, base + selx, neg1)
        lane = lax.iota(_i32, L) + g * L
        vbuf[pl.ds(g * L, L)] = jnp.where(lane < take_w, 1.0, 0.0)
        return run_sel + jnp.sum(seli), run_eq + jnp.sum(eqi)

    pl.loop(0, S // L, init_carry=(jnp.int32(0), jnp.int32(0)))(_sel)

    @pl.when(cid == 0)
    def _wvalid():
        pltpu.sync_copy(vbuf, valid.at[pl.ds(base, S)])

    pltpu.sync_copy(idxbuf, idx_sh.at[pl.ds(base, S)])

    NRG = D // RG // NC

    def _row0(p):
        return (jnp.int32(NC) * p + cid) * RG

    def _fire_loads(p, half):
        r0 = _row0(p)
        for r in range(RG):
            pltpu.async_copy(hT.at[pl.ds((r0 + r) * NP + base, S)],
                             hbuf.at[pl.ds((half * RG + r) * S, S)], lsem)

    def _drain(sem, n):
        for _ in range(n):
            pltpu.make_async_copy(hT.at[pl.ds(0, S)],
                                  hbuf.at[pl.ds(0, S)], sem).wait()

    _fire_loads(jnp.int32(0), 0)
    for p in range(NRG):
        half = p % 2
        if p + 1 < NRG:
            _fire_loads(jnp.int32(p + 1), 1 - half)
        _drain(lsem, RG)
        if p >= 2:
            _drain(ssem, RG)

        @pl.loop(0, S // L)
        def _cmp(g, half=half):
            m = mbuf[pl.ds(g * L, L)] > 0
            pos = posbuf[pl.ds(g * L, L)][0]
            v = svals[pl.ds(g * L, L)]
            for r in range(RG):
                hv = hbuf[pl.ds((half * RG + r) * S + g * L, L)] * v
                plsc.store_compressed(
                    obuf.at[pl.ds((half * RG + r) * (S + L) + pos, L)], hv,
                    mask=m)

        r0 = _row0(p)
        for r in range(RG):
            pltpu.async_copy(obuf.at[pl.ds((half * RG + r) * (S + L), S)],
                             xnT.at[pl.ds((r0 + r) * NP + base, S)], ssem)
    _drain(ssem, 2 * RG)

    plsc.subcore_barrier()
    pltpu.sync_copy(idx_sh, idxfull)

    if remap:
        ein = (src, dst)
        eout = (src2, dst2)
        nech = ET // ECH
        for c in range(NC):
            @pl.when(cid == c)
            def _remap(c=c):
                pltpu.async_copy(ein[c].at[pl.ds(w * ET, ECH)],
                                 ebuf.at[pl.ds(0, ECH)], lsem)
                for i in range(nech):
                    half = i % 2
                    if i + 1 < nech:
                        pltpu.async_copy(
                            ein[c].at[pl.ds(w * ET + (i + 1) * ECH, ECH)],
                            ebuf.at[pl.ds((1 - half) * ECH, ECH)], lsem)
                    pltpu.make_async_copy(ein[c].at[pl.ds(0, ECH)],
                                          ebuf.at[pl.ds(0, ECH)], lsem).wait()
                    if i >= 2:
                        pltpu.make_async_copy(
                            ein[c].at[pl.ds(0, ECH)],
                            ebuf.at[pl.ds(0, ECH)], ssem).wait()

                    @pl.loop(0, ECH // L)
                    def _egrp(g, half=half):
                        ev = ebuf[pl.ds(half * ECH + g * L, L)]
                        got = plsc.load_gather(idxfull, [jnp.maximum(ev, 0)])
                        rbuf[pl.ds(half * ECH + g * L, L)] = jnp.where(
                            ev >= 0, got, neg1)

                    pltpu.async_copy(rbuf.at[pl.ds(half * ECH, ECH)],
                                     eout[c].at[pl.ds(w * ET + i * ECH, ECH)],
                                     ssem)
                pltpu.make_async_copy(ein[c].at[pl.ds(0, ECH)],
                                      ebuf.at[pl.ds(0, ECH)], ssem).wait()
                pltpu.make_async_copy(ein[c].at[pl.ds(0, ECH)],
                                      ebuf.at[pl.ds(0, ECH)], ssem).wait()


def _make_topk(K, remap):
    mesh = plsc.VectorSubcoreMesh(core_axis_name="c", subcore_axis_name="s",
                                  num_cores=NC, num_subcores=NS)
    outs = [jax.ShapeDtypeStruct((D * NP,), _f32),
            jax.ShapeDtypeStruct((NP,), _f32)]
    if remap:
        outs += [jax.ShapeDtypeStruct((E,), _i32),
                 jax.ShapeDtypeStruct((E,), _i32)]
    return pl.kernel(
        functools.partial(_topk_body, K, remap),
        out_type=tuple(outs),
        mesh=mesh,
        scratch_types=[
            pltpu.VMEM((S,), _f32),
            pltpu.VMEM((S,), _i32),
            pltpu.VMEM((S,), _i32),
            pltpu.VMEM((S,), _i32),
            pltpu.VMEM((S,), _i32),
            pltpu.VMEM((S,), _f32),
            pltpu.VMEM((L,), _i32),
            pltpu.VMEM((2 * NS * L,), _i32),
            pltpu.VMEM((NP,), _i32),
            pltpu.VMEM((2 * RG * S,), _f32),
            pltpu.VMEM((2 * RG * (S + L),), _f32),
            pltpu.VMEM((2 * ECH,), _i32),
            pltpu.VMEM((2 * ECH,), _i32),
            pltpu.VMEM((256,), _i32),
            pltpu.VMEM((NS * 256,), _i32),
            pltpu.VMEM((256,), _i32),
            pltpu.SemaphoreType.DMA,
            pltpu.SemaphoreType.DMA,
            pltpu.VMEM_SHARED((2 * NS * L,), _i32),
            pltpu.VMEM_SHARED((NP,), _i32),
            pltpu.VMEM_SHARED((NS * 256,), _i32),
        ],
        compiler_params=pltpu.CompilerParams(needs_layout_passes=False),
        name="topk_sc",
    )


def _final_body(K3, xT, valid, W4T, b4, W5T, b5, out):
    xm = xT[...] * valid[...]
    g = jnp.sum(xm, axis=1, keepdims=True) / K3
    h = jnp.dot(W4T[...], g, preferred_element_type=_f32) + b4[...]
    h = jnp.maximum(h, 0.0)
    z = jnp.dot(W5T[...], h, preferred_element_type=_f32) + b5[...]
    m = jnp.max(z, axis=0, keepdims=True)
    e = jnp.exp(z - m)
    lse = jnp.log(jnp.sum(e, axis=0, keepdims=True)) + m
    out[...] = z - lse


def _make_final(K3):
    return pl.pallas_call(
        functools.partial(_final_body, float(K3)),
        out_shape=jax.ShapeDtypeStruct((10, 1), _f32),
        name="pool_mlp_tc",
    )


def kernel(x, edge_index, batch, Wl1, bl1, Wr1, p1, Wl2, bl2, Wr2, p2,
           Wl3, bl3, Wr3, p3, W4, b4, W5, b5):
    del batch
    n = x.shape[0]
    xT = jnp.pad(x.T, ((0, 0), (0, NP - n))).reshape(-1)
    src = edge_index[0]
    dst = edge_index[1]
    valid = jnp.pad(jnp.ones((n,), _f32), (0, NP - n))

    agg = _make_agg()
    dense = _make_dense()
    ks = [int(np.ceil(0.8 * n))]
    ks.append(int(np.ceil(0.8 * ks[0])))
    ks.append(int(np.ceil(0.8 * ks[1])))

    layers = [(Wl1, bl1, Wr1, p1), (Wl2, bl2, Wr2, p2), (Wl3, bl3, Wr3, p3)]
    for i, (Wl, bl, Wr, p) in enumerate(layers):
        aggT, cnt = agg(xT, src, dst)
        ph = (p / jnp.linalg.norm(p)).reshape(1, D)
        hT, s = dense(aggT.reshape(D, NP), cnt.reshape(1, NP),
                      xT.reshape(D, NP), valid.reshape(1, NP),
                      Wl.T, bl.reshape(D, 1), Wr.T, ph)
        remap = i < 2
        tk = _make_topk(ks[i], remap)
        if remap:
            xT, valid, src, dst = tk(s.reshape(NP), hT.reshape(-1), src, dst)
        else:
            xT, valid = tk(s.reshape(NP), hT.reshape(-1), src, dst)

    out = _make_final(ks[2])(xT.reshape(D, NP), valid.reshape(1, NP), W4.T,
                             b4.reshape(64, 1), W5.T, b5.reshape(10, 1))
    return out.reshape(1, 10)


# speedup vs baseline: 1.2204x; 1.0450x over previous
import functools

import jax
import jax.numpy as jnp
import numpy as np
from jax import lax
from jax.experimental import pallas as pl
from jax.experimental.pallas import tpu as pltpu
from jax.experimental.pallas import tpu_sc as plsc

NC, NS, L = 2, 16, 16
NW = NC * NS
D = 128
N0 = 10000
E = 320000
NP = 10240
S = NP // NS
RPT = D // NW
ECH = 4000
ECHA = 8000
ET = E // NS
RG = 16
CNT_W = NW - 1

_i32 = jnp.int32
_f32 = jnp.float32


def _skey_const(x):
    b = np.float32(x).view(np.int32)
    return int(b) if b >= 0 else int(np.int32(-2147483648) - b)


_LO0 = _skey_const(-2.5)
_HI0 = _skey_const(1.5)


def _skey(v):
    b = plsc.bitcast(v, _i32)
    return jnp.where(b >= 0, b, jnp.int32(-2147483648) - b)


def _agg_body(xT, src, dst, aggT, cnt, xrows, accr, cntb, sb0, db0, sb1, db1,
              sem0, sem1):
    cid = lax.axis_index("c")
    sid = lax.axis_index("s")
    w = sid * NC + cid
    c0 = w * RPT

    for r in range(RPT):
        pltpu.sync_copy(xT.at[pl.ds((c0 + r) * NP, NP)],
                        xrows.at[pl.ds(r * NP, NP)])

    zf = jnp.zeros((L,), _f32)

    @pl.loop(0, RPT * NP // L)
    def _zero(j):
        accr[pl.ds(j * L, L)] = zf

    @pl.when(w == CNT_W)
    def _zero_cnt():
        @pl.loop(0, NP // L)
        def _z2(j):
            cntb[pl.ds(j * L, L)] = zf

    nchunks = E // ECHA
    sbufs = (sb0, sb1)
    dbufs = (db0, db1)
    sems = (sem0, sem1)

    cp0 = pltpu.async_copy(src.at[pl.ds(0, ECHA)], sb0, sem0)
    cp0b = pltpu.async_copy(dst.at[pl.ds(0, ECHA)], db0, sem0)
    cp1 = pltpu.async_copy(src.at[pl.ds(ECHA, ECHA)], sb1, sem1)
    cp1b = pltpu.async_copy(dst.at[pl.ds(ECHA, ECHA)], db1, sem1)

    ones = jnp.ones((L,), _f32)

    @pl.loop(0, nchunks)
    def _chunk(i):
        b = lax.rem(i, 2)
        for bi in range(2):
            @pl.when(b == bi)
            def _proc():
                sbuf, dbuf, sem = sbufs[bi], dbufs[bi], sems[bi]
                pltpu.make_async_copy(src.at[pl.ds(0, ECHA)], sbuf, sem).wait()
                pltpu.make_async_copy(dst.at[pl.ds(0, ECHA)], dbuf, sem).wait()

                @functools.partial(plsc.parallel_loop, 0, ECHA // L, unroll=8)
                def _grp(g):
                    s16 = sbuf[pl.ds(g * L, L)]
                    d16 = dbuf[pl.ds(g * L, L)]
                    m = (s16 >= 0) & (d16 >= 0)
                    sc = jnp.maximum(s16, 0)
                    dc = jnp.maximum(d16, 0)
                    for r in range(RPT):
                        g1 = plsc.load_gather(xrows, [sc + (r * NP)])
                        plsc.addupdate_scatter(accr, [dc + (r * NP)], g1,
                                               mask=m)

                @pl.when(w == CNT_W)
                def _cnt():
                    @functools.partial(plsc.parallel_loop, 0, ECHA // L,
                                       unroll=4)
                    def _cgrp(g):
                        s16 = sbuf[pl.ds(g * L, L)]
                        d16 = dbuf[pl.ds(g * L, L)]
                        m = (s16 >= 0) & (d16 >= 0)
                        dc = jnp.maximum(d16, 0)
                        plsc.addupdate_scatter(cntb, [dc], ones, mask=m)

                @pl.when(i + 2 < nchunks)
                def _refill():
                    off = (i + 2) * ECHA
                    pltpu.async_copy(src.at[pl.ds(off, ECHA)], sbuf, sem)
                    pltpu.async_copy(dst.at[pl.ds(off, ECHA)], dbuf, sem)

    for r in range(RPT):
        pltpu.sync_copy(accr.at[pl.ds(r * NP, NP)],
                        aggT.at[pl.ds((c0 + r) * NP, NP)])

    @pl.when(w == CNT_W)
    def _wcnt():
        pltpu.sync_copy(cntb, cnt)


def _make_agg():
    mesh = plsc.VectorSubcoreMesh(core_axis_name="c", subcore_axis_name="s",
                                  num_cores=NC, num_subcores=NS)
    return pl.kernel(
        _agg_body,
        out_type=(jax.ShapeDtypeStruct((D * NP,), _f32),
                  jax.ShapeDtypeStruct((NP,), _f32)),
        mesh=mesh,
        scratch_types=[
            pltpu.VMEM((RPT * NP,), _f32),
            pltpu.VMEM((RPT * NP,), _f32),
            pltpu.VMEM((NP,), _f32),
            pltpu.VMEM((ECHA,), _i32),
            pltpu.VMEM((ECHA,), _i32),
            pltpu.VMEM((ECHA,), _i32),
            pltpu.VMEM((ECHA,), _i32),
            pltpu.SemaphoreType.DMA,
            pltpu.SemaphoreType.DMA,
        ],
        compiler_params=pltpu.CompilerParams(needs_layout_passes=False),
        name="sage_agg_sc",
    )


def _dense_body(aggT, cnt, xT, valid, WlT, bl, WrT, ph, hT, s):
    rcp = 1.0 / jnp.maximum(cnt[...], 1.0)
    mean = aggT[...] * rcp
    h = jnp.dot(WlT[...], mean, preferred_element_type=_f32)
    h = h + jnp.dot(WrT[...], xT[...], preferred_element_type=_f32)
    h = jnp.maximum(h + bl[...], 0.0)
    hT[...] = h
    sc = jnp.tanh(jnp.dot(ph[...], h, preferred_element_type=_f32))
    s[...] = jnp.where(valid[...] > 0.0, sc, -2.0)


def _make_dense(blk=512):
    grid = (NP // blk,)
    full = pl.BlockSpec((D, D), lambda i: (0, 0))
    colv = pl.BlockSpec((1, blk), lambda i: (0, i))
    mat = pl.BlockSpec((D, blk), lambda i: (0, i))
    return pl.pallas_call(
        _dense_body,
        grid=grid,
        in_specs=[mat, colv, mat, colv, full,
                  pl.BlockSpec((D, 1), lambda i: (0, 0)), full,
                  pl.BlockSpec((1, D), lambda i: (0, 0))],
        out_specs=[mat, colv],
        out_shape=(jax.ShapeDtypeStruct((D, NP), _f32),
                   jax.ShapeDtypeStruct((1, NP), _f32)),
        name="sage_dense_tc",
    )


def _topk_body(K, remap, s_in, hT, src, dst, *rest):
    if remap:
        (xnT, valid, src2, dst2, svals, keys, mbuf, posbuf, idxbuf,
         vbuf, cntrow, cntv, idxfull, hbuf, obuf, ebuf, rbuf, hist,
         hmerge, hsum, lsem, ssem, cnt_sh, idx_sh, hist_sh) = rest
    else:
        (xnT, valid, svals, keys, mbuf, posbuf, idxbuf,
         vbuf, cntrow, cntv, idxfull, hbuf, obuf, ebuf, rbuf, hist,
         hmerge, hsum, lsem, ssem, cnt_sh, idx_sh, hist_sh) = rest
        src2 = dst2 = None

    cid = lax.axis_index("c")
    w = lax.axis_index("s")
    base = w * S

    pltpu.sync_copy(s_in.at[pl.ds(base, S)], svals)

    @pl.loop(0, S // L)
    def _keys(j):
        keys[pl.ds(j * L, L)] = _skey(svals[pl.ds(j * L, L)])

    def _count_ge(t):
        tv = jnp.full((L,), t, _i32)

        def _acc(j, a):
            return a + jnp.where(keys[pl.ds(j * L, L)] >= tv, 1, 0)

        acc = pl.loop(0, S // L, init_carry=jnp.zeros((L,), _i32))(_acc)
        return jnp.sum(acc)

    MIN32 = jnp.int32(-2147483648)
    ones_i = jnp.ones((L,), _i32)
    zi = jnp.zeros((L,), _i32)
    kk = jnp.int32(K)
    pfx = jnp.int32(0)
    for rnd, shift in enumerate((24, 16, 8, 0)):
        for c in range(256 // L):
            hist[pl.ds(c * L, L)] = zi
        sh8 = shift + 8

        @pl.loop(0, S // L)
        def _hloc(g, rnd=rnd, shift=shift, sh8=sh8, pfx=pfx):
            kc = keys[pl.ds(g * L, L)]
            uk = kc ^ MIN32
            byte = lax.shift_right_logical(uk, jnp.int32(shift)) & 0xFF
            if rnd == 0:
                plsc.addupdate_scatter(hist, [byte], ones_i)
            else:
                mm = lax.shift_right_logical(uk, jnp.int32(sh8)) == pfx
                plsc.addupdate_scatter(hist, [byte], ones_i, mask=mm)

        pltpu.sync_copy(hist, hist_sh.at[pl.ds(w * 256, 256)])
        plsc.subcore_barrier()
        pltpu.sync_copy(hist_sh, hmerge)
        for c in range(256 // L):
            acc = hmerge[pl.ds(c * L, L)]
            for t in range(1, NS):
                acc = acc + hmerge[pl.ds(t * 256 + c * L, L)]
            hsum[pl.ds(c * L, L)] = acc
        plsc.subcore_barrier()

        found = jnp.int32(0)
        bsel = jnp.int32(0)
        above = jnp.int32(0)
        for c in range(256 // L - 1, -1, -1):
            hv = hsum[pl.ds(c * L, L)]
            rv = lax.rev(hv, (0,))
            inc = plsc.cumsum(rv)
            sel = (inc + above) >= kk
            npos = plsc.all_reduce_population_count(sel)
            ffs = plsc.all_reduce_ffs(sel)
            npos = npos[0] if getattr(npos, "ndim", 0) else npos
            ffs = ffs[0] if getattr(ffs, "ndim", 0) else ffs
            hit = (found == 0) & (npos > 0)
            bsel = jnp.where(hit, jnp.int32(c * L + (L - 1)) - ffs, bsel)
            found = jnp.where(npos > 0, jnp.int32(1), found)
            above = above + inc[L - 1]
        cgt = zi
        for c in range(256 // L):
            binv = lax.iota(_i32, L) + c * L
            cgt = cgt + jnp.where(binv > bsel, hsum[pl.ds(c * L, L)], 0)
        kk = kk - jnp.sum(cgt)
        pfx = lax.shift_left(pfx, jnp.int32(8)) | bsel
    tstar = pfx ^ MIN32

    cg = _count_ge(tstar + 1)
    ce = _count_ge(tstar) - cg
    cntrow[...] = jnp.full((L,), cg, _i32)
    pltpu.sync_copy(cntrow, cnt_sh.at[pl.ds(w * L, L)])
    cntrow[...] = jnp.full((L,), ce, _i32)
    pltpu.sync_copy(cntrow, cnt_sh.at[pl.ds((NS + w) * L, L)])
    plsc.subcore_barrier()
    pltpu.sync_copy(cnt_sh, cntv)
    plsc.subcore_barrier()

    c_gt_tot = cntv[pl.ds(0, L)][0]
    for u in range(1, NS):
        c_gt_tot = c_gt_tot + cntv[pl.ds(u * L, L)][0]
    eq_before = jnp.int32(0)
    for u in range(NS):
        eq_before = jnp.where(u < w, eq_before + cntv[pl.ds((NS + u) * L, L)][0],
                              eq_before)
    r_need = K - c_gt_tot
    ce_w = cntv[pl.ds((NS + w.astype(_i32)) * L, L)][0]
    take_eq = jnp.clip(r_need - eq_before, 0, ce_w)
    take_w = cntv[pl.ds(w * L, L)][0] + take_eq

    tv = jnp.full((L,), tstar, _i32)
    neg1 = jnp.full((L,), -1, _i32)

    def _sel(g, carry):
        run_sel, run_eq = carry
        kc = keys[pl.ds(g * L, L)]
        gt = kc > tv
        eq = kc == tv
        eqi = jnp.where(eq, 1, 0)
        eqx = plsc.cumsum(eqi) - eqi + run_eq
        sel = gt | (eq & (eqx < take_eq))
        seli = jnp.where(sel, 1, 0)
        selx = plsc.cumsum(seli) - seli + run_sel
        mbuf[pl.ds(g * L, L)] = seli
        posbuf[pl.ds(g * L, L)] = jnp.full((L,), run_sel, _i32)
        idxbuf[pl.ds(g * L, L)] = jnp.where(sel, base + selx, neg1)
        lane = lax.iota(_i32, L) + g * L
        vbuf[pl.ds(g * L, L)] = jnp.where(lane < take_w, 1.0, 0.0)
        return run_sel + jnp.sum(seli), run_eq + jnp.sum(eqi)

    pl.loop(0, S // L, init_carry=(jnp.int32(0), jnp.int32(0)))(_sel)

    @pl.when(cid == 0)
    def _wvalid():
        pltpu.sync_copy(vbuf, valid.at[pl.ds(base, S)])

    pltpu.sync_copy(idxbuf, idx_sh.at[pl.ds(base, S)])

    NRG = D // RG // NC

    def _row0(p):
        return (jnp.int32(NC) * p + cid) * RG

    def _fire_loads(p, half):
        r0 = _row0(p)
        for r in range(RG):
            pltpu.async_copy(hT.at[pl.ds((r0 + r) * NP + base, S)],
                             hbuf.at[pl.ds((half * RG + r) * S, S)], lsem)

    def _drain(sem, n):
        for _ in range(n):
            pltpu.make_async_copy(hT.at[pl.ds(0, S)],
                                  hbuf.at[pl.ds(0, S)], sem).wait()

    _fire_loads(jnp.int32(0), 0)
    for p in range(NRG):
        half = p % 2
        if p + 1 < NRG:
            _fire_loads(jnp.int32(p + 1), 1 - half)
        _drain(lsem, RG)
        if p >= 2:
            _drain(ssem, RG)

        @pl.loop(0, S // L)
        def _cmp(g, half=half):
            m = mbuf[pl.ds(g * L, L)] > 0
            pos = posbuf[pl.ds(g * L, L)][0]
            v = svals[pl.ds(g * L, L)]
            for r in range(RG):
                hv = hbuf[pl.ds((half * RG + r) * S + g * L, L)] * v
                plsc.store_compressed(
                    obuf.at[pl.ds((half * RG + r) * (S + L) + pos, L)], hv,
                    mask=m)

        r0 = _row0(p)
        for r in range(RG):
            pltpu.async_copy(obuf.at[pl.ds((half * RG + r) * (S + L), S)],
                             xnT.at[pl.ds((r0 + r) * NP + base, S)], ssem)
    _drain(ssem, 2 * RG)

    plsc.subcore_barrier()
    pltpu.sync_copy(idx_sh, idxfull)

    if remap:
        ein = (src, dst)
        eout = (src2, dst2)
        nech = ET // ECH
        for c in range(NC):
            @pl.when(cid == c)
            def _remap(c=c):
                pltpu.async_copy(ein[c].at[pl.ds(w * ET, ECH)],
                                 ebuf.at[pl.ds(0, ECH)], lsem)
                for i in range(nech):
                    half = i % 2
                    if i + 1 < nech:
                        pltpu.async_copy(
                            ein[c].at[pl.ds(w * ET + (i + 1) * ECH, ECH)],
                            ebuf.at[pl.ds((1 - half) * ECH, ECH)], lsem)
                    pltpu.make_async_copy(ein[c].at[pl.ds(0, ECH)],
                                          ebuf.at[pl.ds(0, ECH)], lsem).wait()
                    if i >= 2:
                        pltpu.make_async_copy(
                            ein[c].at[pl.ds(0, ECH)],
                            ebuf.at[pl.ds(0, ECH)], ssem).wait()

                    @pl.loop(0, ECH // L)
                    def _egrp(g, half=half):
                        ev = ebuf[pl.ds(half * ECH + g * L, L)]
                        got = plsc.load_gather(idxfull, [jnp.maximum(ev, 0)])
                        rbuf[pl.ds(half * ECH + g * L, L)] = jnp.where(
                            ev >= 0, got, neg1)

                    pltpu.async_copy(rbuf.at[pl.ds(half * ECH, ECH)],
                                     eout[c].at[pl.ds(w * ET + i * ECH, ECH)],
                                     ssem)
                pltpu.make_async_copy(ein[c].at[pl.ds(0, ECH)],
                                      ebuf.at[pl.ds(0, ECH)], ssem).wait()
                pltpu.make_async_copy(ein[c].at[pl.ds(0, ECH)],
                                      ebuf.at[pl.ds(0, ECH)], ssem).wait()


def _make_topk(K, remap):
    mesh = plsc.VectorSubcoreMesh(core_axis_name="c", subcore_axis_name="s",
                                  num_cores=NC, num_subcores=NS)
    outs = [jax.ShapeDtypeStruct((D * NP,), _f32),
            jax.ShapeDtypeStruct((NP,), _f32)]
    if remap:
        outs += [jax.ShapeDtypeStruct((E,), _i32),
                 jax.ShapeDtypeStruct((E,), _i32)]
    return pl.kernel(
        functools.partial(_topk_body, K, remap),
        out_type=tuple(outs),
        mesh=mesh,
        scratch_types=[
            pltpu.VMEM((S,), _f32),
            pltpu.VMEM((S,), _i32),
            pltpu.VMEM((S,), _i32),
            pltpu.VMEM((S,), _i32),
            pltpu.VMEM((S,), _i32),
            pltpu.VMEM((S,), _f32),
            pltpu.VMEM((L,), _i32),
            pltpu.VMEM((2 * NS * L,), _i32),
            pltpu.VMEM((NP,), _i32),
            pltpu.VMEM((2 * RG * S,), _f32),
            pltpu.VMEM((2 * RG * (S + L),), _f32),
            pltpu.VMEM((2 * ECH,), _i32),
            pltpu.VMEM((2 * ECH,), _i32),
            pltpu.VMEM((256,), _i32),
            pltpu.VMEM((NS * 256,), _i32),
            pltpu.VMEM((256,), _i32),
            pltpu.SemaphoreType.DMA,
            pltpu.SemaphoreType.DMA,
            pltpu.VMEM_SHARED((2 * NS * L,), _i32),
            pltpu.VMEM_SHARED((NP,), _i32),
            pltpu.VMEM_SHARED((NS * 256,), _i32),
        ],
        compiler_params=pltpu.CompilerParams(needs_layout_passes=False),
        name="topk_sc",
    )


def _final_body(K3, xT, valid, W4T, b4, W5T, b5, out):
    xm = xT[...] * valid[...]
    g = jnp.sum(xm, axis=1, keepdims=True) / K3
    h = jnp.dot(W4T[...], g, preferred_element_type=_f32) + b4[...]
    h = jnp.maximum(h, 0.0)
    z = jnp.dot(W5T[...], h, preferred_element_type=_f32) + b5[...]
    m = jnp.max(z, axis=0, keepdims=True)
    e = jnp.exp(z - m)
    lse = jnp.log(jnp.sum(e, axis=0, keepdims=True)) + m
    out[...] = z - lse


def _make_final(K3):
    return pl.pallas_call(
        functools.partial(_final_body, float(K3)),
        out_shape=jax.ShapeDtypeStruct((10, 1), _f32),
        name="pool_mlp_tc",
    )


def kernel(x, edge_index, batch, Wl1, bl1, Wr1, p1, Wl2, bl2, Wr2, p2,
           Wl3, bl3, Wr3, p3, W4, b4, W5, b5):
    del batch
    n = x.shape[0]
    xT = jnp.pad(x.T, ((0, 0), (0, NP - n))).reshape(-1)
    src = edge_index[0]
    dst = edge_index[1]
    valid = jnp.pad(jnp.ones((n,), _f32), (0, NP - n))

    agg = _make_agg()
    dense = _make_dense()
    ks = [int(np.ceil(0.8 * n))]
    ks.append(int(np.ceil(0.8 * ks[0])))
    ks.append(int(np.ceil(0.8 * ks[1])))

    layers = [(Wl1, bl1, Wr1, p1), (Wl2, bl2, Wr2, p2), (Wl3, bl3, Wr3, p3)]
    for i, (Wl, bl, Wr, p) in enumerate(layers):
        aggT, cnt = agg(xT, src, dst)
        ph = (p / jnp.linalg.norm(p)).reshape(1, D)
        hT, s = dense(aggT.reshape(D, NP), cnt.reshape(1, NP),
                      xT.reshape(D, NP), valid.reshape(1, NP),
                      Wl.T, bl.reshape(D, 1), Wr.T, ph)
        remap = i < 2
        tk = _make_topk(ks[i], remap)
        if remap:
            xT, valid, src, dst = tk(s.reshape(NP), hT.reshape(-1), src, dst)
        else:
            xT, valid = tk(s.reshape(NP), hT.reshape(-1), src, dst)

    out = _make_final(ks[2])(xT.reshape(D, NP), valid.reshape(1, NP), W4.T,
                             b4.reshape(64, 1), W5.T, b5.reshape(10, 1))
    return out.reshape(1, 10)
